# Initial kernel scaffold; baseline (speedup 1.0000x reference)
import jax
import jax.numpy as jnp
from jax.experimental import pallas as pl


def kernel(x, edge_index, W_pre, b_pre, W_rel0, b_rel0, W_root0, W_rel1, b_rel1, W_root1, W_post, b_post):
    raise NotImplementedError("write your pallas kernel here")


# speedup vs baseline: 1.1273x; 1.1273x over previous
import functools

import jax
import jax.numpy as jnp
from jax import lax
from jax.experimental import pallas as pl
from jax.experimental.pallas import tpu as pltpu
from jax.experimental.pallas import tpu_sc as plsc

_N = 50000
_E = 800000
_H = 64
_RB = 784
_NGRID = 64
_NTILES = 16
_EB = 128
_NBLK = 392
_EPT = _NBLK * _EB
_NCHUNK = 6
_Q = 8448
_ACC = 9472
_DUMMY = _Q
_GB = 56
_NGRP = 7
_ZR = 56
_ZC = 10
_ZTAIL = 32
_OROWS = _Q // _NTILES


def _mk_m128(h, w_rel):
    m = jnp.dot(h, w_rel, preferred_element_type=jnp.float32)
    return jnp.concatenate([m, jnp.zeros_like(m)], axis=1)


def _tc1_body(x_ref, wpre_ref, bpre_ref, wrel_ref, wroot_ref, m_ref, r_ref):
    h = jnp.maximum(
        jnp.dot(x_ref[...], wpre_ref[...], preferred_element_type=jnp.float32)
        + bpre_ref[...], 0.0)
    m_ref[...] = _mk_m128(h, wrel_ref[...])
    r_ref[...] = jnp.dot(h, wroot_ref[...], preferred_element_type=jnp.float32)


def _tc2_body(agg_ref, r_ref, brel_ref, wrel_ref, wroot_ref, m_ref, rout_ref):
    h = jnp.maximum(agg_ref[:, :_H] + brel_ref[...] + r_ref[...], 0.0)
    m_ref[...] = _mk_m128(h, wrel_ref[...])
    rout_ref[...] = jnp.dot(h, wroot_ref[...],
                            preferred_element_type=jnp.float32)


def _tc3_body(agg_ref, r_ref, brel_ref, wpost_ref, bpost_ref, out_ref):
    h = jnp.maximum(agg_ref[:, :_H] + brel_ref[...] + r_ref[...], 0.0)
    out_ref[...] = jnp.maximum(
        jnp.dot(h, wpost_ref[...], preferred_element_type=jnp.float32)
        + bpost_ref[...], 0.0)


def _tc1(x8, w8, bpre, wrel, wroot):
    return pl.pallas_call(
        _tc1_body,
        grid=(_NGRID,),
        in_specs=[
            pl.BlockSpec((_RB, 8), lambda i: (i, 0)),
            pl.BlockSpec((8, _H), lambda i: (0, 0)),
            pl.BlockSpec((1, _H), lambda i: (0, 0)),
            pl.BlockSpec((_H, _H), lambda i: (0, 0)),
            pl.BlockSpec((_H, _H), lambda i: (0, 0)),
        ],
        out_specs=[
            pl.BlockSpec((_RB, 2 * _H), lambda i: (i, 0)),
            pl.BlockSpec((_RB, _H), lambda i: (i, 0)),
        ],
        out_shape=[
            jax.ShapeDtypeStruct((_N, 2 * _H), jnp.float32),
            jax.ShapeDtypeStruct((_N, _H), jnp.float32),
        ],
    )(x8, w8, bpre, wrel, wroot)


def _tc2(agg, r, brel, wrel, wroot):
    return pl.pallas_call(
        _tc2_body,
        grid=(_NGRID,),
        in_specs=[
            pl.BlockSpec((_RB, 2 * _H), lambda i: (i, 0)),
            pl.BlockSpec((_RB, _H), lambda i: (i, 0)),
            pl.BlockSpec((1, _H), lambda i: (0, 0)),
            pl.BlockSpec((_H, _H), lambda i: (0, 0)),
            pl.BlockSpec((_H, _H), lambda i: (0, 0)),
        ],
        out_specs=[
            pl.BlockSpec((_RB, 2 * _H), lambda i: (i, 0)),
            pl.BlockSpec((_RB, _H), lambda i: (i, 0)),
        ],
        out_shape=[
            jax.ShapeDtypeStruct((_N, 2 * _H), jnp.float32),
            jax.ShapeDtypeStruct((_N, _H), jnp.float32),
        ],
    )(agg, r, brel, wrel, wroot)


def _tc3(agg, r, brel, wpost, bpost):
    return pl.pallas_call(
        _tc3_body,
        grid=(_NGRID,),
        in_specs=[
            pl.BlockSpec((_RB, 2 * _H), lambda i: (i, 0)),
            pl.BlockSpec((_RB, _H), lambda i: (i, 0)),
            pl.BlockSpec((1, _H), lambda i: (0, 0)),
            pl.BlockSpec((_H, 2), lambda i: (0, 0)),
            pl.BlockSpec((1, 2), lambda i: (0, 0)),
        ],
        out_specs=[pl.BlockSpec((_RB, 2), lambda i: (i, 0))],
        out_shape=[jax.ShapeDtypeStruct((_N, 2), jnp.float32)],
    )(agg, r, brel, wpost, bpost)


def _seg(m128, src3, dl4):
    mesh = plsc.VectorSubcoreMesh(core_axis_name="c", subcore_axis_name="s")

    @functools.partial(
        pl.kernel,
        out_type=jax.ShapeDtypeStruct((_NCHUNK * _Q, 2 * _H), jnp.float32),
        mesh=mesh,
        scratch_types=[
            pltpu.VMEM((_GB, _EB), jnp.int32),
            pltpu.VMEM((_GB, _EB), jnp.int32),
            pltpu.VMEM((_EB, 2 * _H), jnp.float32),
            pltpu.VMEM((_ZR, 2 * _H), jnp.float32),
            pltpu.VMEM_SHARED((_ACC, 2 * _H), jnp.float32),
        ],
    )
    def seg(m_hbm, src_hbm, dl_hbm, out_hbm, srcv, dstv, rows, zbuf, acc):
        c = lax.axis_index("c")
        s = lax.axis_index("s")

        zvec = jnp.zeros((16,), jnp.float32)

        def _zb(i, carry):
            def _zl(j, carry2):
                zbuf[i, pl.ds(j * 16, 16)] = zvec
                return carry2
            return lax.fori_loop(0, 8, _zl, carry)

        lax.fori_loop(0, _ZR, _zb, 0)

        for p in range(_NCHUNK // 2):
            q = 2 * p + c
            tilebase = s * (_ACC // _NTILES)

            def _za(i, carry):
                pltpu.sync_copy(zbuf, acc.at[pl.ds(tilebase + i * _ZR, _ZR)])
                return carry

            lax.fori_loop(0, _ZC, _za, 0)
            pltpu.sync_copy(zbuf.at[pl.ds(0, _ZTAIL)],
                            acc.at[pl.ds(tilebase + _ZC * _ZR, _ZTAIL)])
            plsc.subcore_barrier()

            def _grp(g, carry):
                pltpu.sync_copy(src_hbm.at[s].at[pl.ds(g * _GB, _GB)], srcv)
                pltpu.sync_copy(dl_hbm.at[q, s].at[pl.ds(g * _GB, _GB)], dstv)

                def _body(b, carry2):
                    pltpu.sync_copy(m_hbm.at[srcv.at[b]], rows)
                    pltpu.sync_copy(rows, acc.at[dstv.at[b]], add=True)
                    return carry2

                return lax.fori_loop(0, _GB, _body, carry)

            lax.fori_loop(0, _NGRP, _grp, 0)
            plsc.subcore_barrier()

            pltpu.sync_copy(
                acc.at[pl.ds(s * _OROWS, _OROWS)],
                out_hbm.at[pl.ds(q * _Q + s * _OROWS, _OROWS)])
            plsc.subcore_barrier()

    return seg(m128, src3, dl4)


def kernel(x, edge_index, W_pre, b_pre, W_rel0, b_rel0, W_root0,
           W_rel1, b_rel1, W_root1, W_post, b_post):
    src = edge_index[0]
    dst = edge_index[1]
    pad = _NTILES * _EPT - _E
    srcp = jnp.concatenate(
        [src, jnp.zeros((pad,), jnp.int32)]).reshape(_NTILES, _NBLK, _EB)
    dstp = jnp.concatenate([dst, jnp.full((pad,), -1, jnp.int32)])
    dls = []
    for qq in range(_NCHUNK):
        base = qq * _Q
        inq = (dstp >= base) & (dstp < base + _Q)
        dls.append(jnp.where(inq, dstp - base,
                             _DUMMY + (dstp & 1023)).astype(jnp.int32))
    dl4 = jnp.stack(dls).reshape(_NCHUNK, _NTILES, _NBLK, _EB)

    x8 = jnp.pad(x, ((0, 0), (0, 5)))
    w8 = jnp.pad(W_pre, ((0, 5), (0, 0)))

    m0, r0 = _tc1(x8, w8, b_pre.reshape(1, _H), W_rel0, W_root0)
    agg0 = _seg(m0, srcp, dl4)
    m1, r1 = _tc2(agg0, r0, b_rel0.reshape(1, _H), W_rel1, W_root1)
    agg1 = _seg(m1, srcp, dl4)
    (out,) = _tc3(agg1, r1, b_rel1.reshape(1, _H), W_post,
                  b_post.reshape(1, 2))
    return out


# speedup vs baseline: 1.6326x; 1.4482x over previous
import functools

import jax
import jax.numpy as jnp
from jax import lax
from jax.experimental import pallas as pl
from jax.experimental.pallas import tpu as pltpu
from jax.experimental.pallas import tpu_sc as plsc

_N = 50000
_E = 800000
_H = 64
_RB = 784
_NGRID = 64
_NTILES = 16
_EB = 128
_NBLK = 392
_EPT = _NBLK * _EB
_NCHUNK = 4
_Q = 12544
_ACC = 12800
_DUMMY = _Q
_GB = 56
_NGRP = 7
_ZR = 32
_ZC = 25
_OROWS = _Q // _NTILES


def _mk_m128(h, w_rel):
    m = jnp.dot(h, w_rel, preferred_element_type=jnp.float32)
    return jnp.concatenate([m, jnp.zeros_like(m)], axis=1)


def _tc1_body(x_ref, wpre_ref, bpre_ref, wrel_ref, wroot_ref, m_ref, r_ref):
    h = jnp.maximum(
        jnp.dot(x_ref[...], wpre_ref[...], preferred_element_type=jnp.float32)
        + bpre_ref[...], 0.0)
    m_ref[...] = _mk_m128(h, wrel_ref[...])
    r_ref[...] = jnp.dot(h, wroot_ref[...], preferred_element_type=jnp.float32)


def _tc2_body(agg_ref, r_ref, brel_ref, wrel_ref, wroot_ref, m_ref, rout_ref):
    h = jnp.maximum(agg_ref[:, :_H] + brel_ref[...] + r_ref[...], 0.0)
    m_ref[...] = _mk_m128(h, wrel_ref[...])
    rout_ref[...] = jnp.dot(h, wroot_ref[...],
                            preferred_element_type=jnp.float32)


def _tc3_body(agg_ref, r_ref, brel_ref, wpost_ref, bpost_ref, out_ref):
    h = jnp.maximum(agg_ref[:, :_H] + brel_ref[...] + r_ref[...], 0.0)
    out_ref[...] = jnp.maximum(
        jnp.dot(h, wpost_ref[...], preferred_element_type=jnp.float32)
        + bpost_ref[...], 0.0)


def _tc1(x8, w8, bpre, wrel, wroot):
    return pl.pallas_call(
        _tc1_body,
        grid=(_NGRID,),
        in_specs=[
            pl.BlockSpec((_RB, 8), lambda i: (i, 0)),
            pl.BlockSpec((8, _H), lambda i: (0, 0)),
            pl.BlockSpec((1, _H), lambda i: (0, 0)),
            pl.BlockSpec((_H, _H), lambda i: (0, 0)),
            pl.BlockSpec((_H, _H), lambda i: (0, 0)),
        ],
        out_specs=[
            pl.BlockSpec((_RB, 2 * _H), lambda i: (i, 0)),
            pl.BlockSpec((_RB, _H), lambda i: (i, 0)),
        ],
        out_shape=[
            jax.ShapeDtypeStruct((_N, 2 * _H), jnp.float32),
            jax.ShapeDtypeStruct((_N, _H), jnp.float32),
        ],
    )(x8, w8, bpre, wrel, wroot)


def _tc2(agg, r, brel, wrel, wroot):
    return pl.pallas_call(
        _tc2_body,
        grid=(_NGRID,),
        in_specs=[
            pl.BlockSpec((_RB, 2 * _H), lambda i: (i, 0)),
            pl.BlockSpec((_RB, _H), lambda i: (i, 0)),
            pl.BlockSpec((1, _H), lambda i: (0, 0)),
            pl.BlockSpec((_H, _H), lambda i: (0, 0)),
            pl.BlockSpec((_H, _H), lambda i: (0, 0)),
        ],
        out_specs=[
            pl.BlockSpec((_RB, 2 * _H), lambda i: (i, 0)),
            pl.BlockSpec((_RB, _H), lambda i: (i, 0)),
        ],
        out_shape=[
            jax.ShapeDtypeStruct((_N, 2 * _H), jnp.float32),
            jax.ShapeDtypeStruct((_N, _H), jnp.float32),
        ],
    )(agg, r, brel, wrel, wroot)


def _tc3(agg, r, brel, wpost, bpost):
    return pl.pallas_call(
        _tc3_body,
        grid=(_NGRID,),
        in_specs=[
            pl.BlockSpec((_RB, 2 * _H), lambda i: (i, 0)),
            pl.BlockSpec((_RB, _H), lambda i: (i, 0)),
            pl.BlockSpec((1, _H), lambda i: (0, 0)),
            pl.BlockSpec((_H, 2), lambda i: (0, 0)),
            pl.BlockSpec((1, 2), lambda i: (0, 0)),
        ],
        out_specs=[pl.BlockSpec((_RB, 2), lambda i: (i, 0))],
        out_shape=[jax.ShapeDtypeStruct((_N, 2), jnp.float32)],
    )(agg, r, brel, wpost, bpost)


def _seg(m128, pk4):
    mesh = plsc.VectorSubcoreMesh(core_axis_name="c", subcore_axis_name="s")

    @functools.partial(
        pl.kernel,
        out_type=jax.ShapeDtypeStruct((_NCHUNK * _Q, 2 * _H), jnp.float32),
        mesh=mesh,
        scratch_types=[
            pltpu.VMEM((_GB, _EB), jnp.int32),
            pltpu.VMEM((_EB,), jnp.int32),
            pltpu.VMEM((_EB,), jnp.int32),
            pltpu.VMEM((_EB, 2 * _H), jnp.float32),
            pltpu.VMEM((_ZR, 2 * _H), jnp.float32),
            pltpu.VMEM_SHARED((_ACC, 2 * _H), jnp.float32),
        ],
    )
    def seg(m_hbm, pk_hbm, out_hbm, pkv, srcv, dstv, rows, zbuf, acc):
        c = lax.axis_index("c")
        s = lax.axis_index("s")

        zvec = jnp.zeros((16,), jnp.float32)

        def _zb(i, carry):
            def _zl(j, carry2):
                zbuf[i, pl.ds(j * 16, 16)] = zvec
                return carry2
            return lax.fori_loop(0, 8, _zl, carry)

        lax.fori_loop(0, _ZR, _zb, 0)

        for p in range(_NCHUNK // 2):
            q = 2 * p + c
            tilebase = s * (_ACC // _NTILES)

            def _za(i, carry):
                pltpu.sync_copy(zbuf, acc.at[pl.ds(tilebase + i * _ZR, _ZR)])
                return carry

            lax.fori_loop(0, _ZC, _za, 0)
            plsc.subcore_barrier()

            def _grp(g, carry):
                pltpu.sync_copy(pk_hbm.at[q, s].at[pl.ds(g * _GB, _GB)], pkv)

                def _body(b, carry2):
                    for kk in range(8):
                        v = pkv[b, pl.ds(kk * 16, 16)]
                        srcv[pl.ds(kk * 16, 16)] = v & 0xFFFF
                        dstv[pl.ds(kk * 16, 16)] = lax.shift_right_logical(
                            v, 16)
                    pltpu.sync_copy(m_hbm.at[srcv], rows)
                    pltpu.sync_copy(rows, acc.at[dstv], add=True)
                    return carry2

                return lax.fori_loop(0, _GB, _body, carry)

            lax.fori_loop(0, _NGRP, _grp, 0)
            plsc.subcore_barrier()

            pltpu.sync_copy(
                acc.at[pl.ds(s * _OROWS, _OROWS)],
                out_hbm.at[pl.ds(q * _Q + s * _OROWS, _OROWS)])
            plsc.subcore_barrier()

    return seg(m128, pk4)


def kernel(x, edge_index, W_pre, b_pre, W_rel0, b_rel0, W_root0,
           W_rel1, b_rel1, W_root1, W_post, b_post):
    src = edge_index[0]
    dst = edge_index[1]
    pad = _NTILES * _EPT - _E
    srcp = jnp.concatenate([src, jnp.zeros((pad,), jnp.int32)])
    dstp = jnp.concatenate([dst, jnp.full((pad,), -1, jnp.int32)])
    pks = []
    for qq in range(_NCHUNK):
        base = qq * _Q
        inq = (dstp >= base) & (dstp < base + _Q)
        dloc = jnp.where(inq, dstp - base,
                         _DUMMY + (dstp & 255)).astype(jnp.int32)
        pks.append(srcp | (dloc << 16))
    pk4 = jnp.stack(pks).reshape(_NCHUNK, _NTILES, _NBLK, _EB)

    x8 = jnp.pad(x, ((0, 0), (0, 5)))
    w8 = jnp.pad(W_pre, ((0, 5), (0, 0)))

    m0, r0 = _tc1(x8, w8, b_pre.reshape(1, _H), W_rel0, W_root0)
    agg0 = _seg(m0, pk4)
    m1, r1 = _tc2(agg0, r0, b_rel0.reshape(1, _H), W_rel1, W_root1)
    agg1 = _seg(m1, pk4)
    (out,) = _tc3(agg1, r1, b_rel1.reshape(1, _H), W_post,
                  b_post.reshape(1, 2))
    return out
